# Initial kernel scaffold; baseline (speedup 1.0000x reference)
import functools

import jax
import jax.numpy as jnp
from jax import lax
from jax.experimental import pallas as pl
from jax.experimental.pallas import tpu as pltpu
from jax.experimental.pallas import tpu_sc as plsc

B, L = 1024, 200
S = 8
C = 16
TOK = B * L
ROWS = TOK // S
D_OUT = 128


_NW = 32
_CHUNK = 128
_TPW = TOK // _NW
_CPW = _TPW // _CHUNK


def _sc_gather_body(table_hbm, idx_hbm, out_hbm, idx_v, rows_v, sem):
    wid = lax.axis_index("s") * 2 + lax.axis_index("c")
    base = wid * _CPW
    pltpu.sync_copy(idx_hbm.at[pl.ds(base, _CPW)], idx_v)
    descs = []
    for i in range(_CPW):
        descs.append(pltpu.async_copy(table_hbm.at[idx_v.at[i]], rows_v.at[i], sem))
    for d in descs:
        d.wait()
    pltpu.sync_copy(rows_v, out_hbm.at[pl.ds(base, _CPW)])


def _sc_gather(table, idx2d):
    mesh = plsc.VectorSubcoreMesh(core_axis_name="c", subcore_axis_name="s")
    kern = functools.partial(
        pl.kernel,
        mesh=mesh,
        out_type=jax.ShapeDtypeStruct((TOK // _CHUNK, _CHUNK, C), jnp.float32),
        scratch_types=[
            pltpu.VMEM((_CPW, _CHUNK), jnp.int32),
            pltpu.VMEM((_CPW, _CHUNK, C), jnp.float32),
            pltpu.SemaphoreType.DMA,
        ],
    )(_sc_gather_body)
    return kern(table, idx2d)


_GRID = 32
_RB = ROWS // _GRID


def _tc_body(xv_ref, vid_ref, did_ref, p0_ref, p1_ref, p2_ref,
             de_ref, s0_ref, s1_ref, s2_ref, e8_ref, wt_ref, b_ref, out_ref):
    f32 = jnp.float32
    hi = jax.lax.Precision.HIGHEST
    mvec = (vid_ref[...] != 0).astype(f32)
    mask = jax.lax.dot(mvec, e8_ref[...], precision=hi)
    x = xv_ref[...] * mask
    i8 = lax.broadcasted_iota(jnp.int32, (_RB, 8), 1)
    i128 = lax.broadcasted_iota(jnp.int32, (_RB, 128), 1)
    pieces = []
    for k in range(S):
        ohd = (did_ref[:, k:k + 1] == i8).astype(f32)
        acc = jax.lax.dot(ohd, de_ref[...], precision=hi)
        for p_ref, s_ref in ((p0_ref, s0_ref), (p1_ref, s1_ref), (p2_ref, s2_ref)):
            ohp = (p_ref[:, k:k + 1] == i128).astype(f32)
            acc = acc + jax.lax.dot(ohp, s_ref[...], precision=hi)
        pieces.append(acc)
    x = x + jnp.concatenate(pieces, axis=1)
    y = jax.lax.dot(x, wt_ref[...], precision=hi) + b_ref[...]
    out_ref[...] = y


def _tc_embed_conv(xv, vid, did, p0, p1, p2, de8, se_z, e8, wt, bias):
    def full(shape):
        return pl.BlockSpec(shape, lambda *_: tuple(0 for _ in shape))

    return pl.pallas_call(
        _tc_body,
        grid=(_GRID,),
        in_specs=[
            pl.BlockSpec((_RB, D_OUT), lambda i: (i, 0)),
            pl.BlockSpec((_RB, S), lambda i: (i, 0)),
            pl.BlockSpec((_RB, S), lambda i: (i, 0)),
            pl.BlockSpec((_RB, S), lambda i: (i, 0)),
            pl.BlockSpec((_RB, S), lambda i: (i, 0)),
            pl.BlockSpec((_RB, S), lambda i: (i, 0)),
            full((S, C)),
            full((2 * 64, C)),
            full((2 * 64, C)),
            full((2 * 64, C)),
            full((S, D_OUT)),
            full((S * C, D_OUT)),
            full((1, D_OUT)),
        ],
        out_specs=pl.BlockSpec((_RB, D_OUT), lambda i: (i, 0)),
        out_shape=jax.ShapeDtypeStruct((ROWS, D_OUT), jnp.float32),
    )(xv, vid, did, p0, p1, p2, de8, se_z[0], se_z[1], se_z[2], e8, wt, bias)


def kernel(value, depth, position, tgt_value_emb, tgt_depth_emb,
           tgt_spatial_emb, conv_w, conv_b):
    value = value.astype(jnp.int32)
    depth = depth.astype(jnp.int32)
    position = position.astype(jnp.int32)

    idx2d = value.reshape(TOK // _CHUNK, _CHUNK)
    xv3 = _sc_gather(tgt_value_emb, idx2d)
    xv = xv3.reshape(ROWS, D_OUT)

    de8 = jnp.zeros((S, C), jnp.float32).at[1:7].set(tgt_depth_emb[1:])
    se_z = tgt_spatial_emb.at[:, 0, :].set(0.0)

    vid = value.reshape(ROWS, S)
    did = depth.reshape(ROWS, S)
    pos = position.reshape(ROWS, S, 3)
    p0, p1, p2 = pos[:, :, 0], pos[:, :, 1], pos[:, :, 2]

    e8 = jnp.repeat(jnp.eye(S, dtype=jnp.float32), C, axis=1)
    wt = conv_w.transpose(2, 1, 0).reshape(S * C, D_OUT)
    bias = conv_b.reshape(1, D_OUT)

    out = _tc_embed_conv(xv, vid, did, p0, p1, p2, de8, se_z, e8, wt, bias)
    return out.reshape(B, L // S, D_OUT)


# speedup vs baseline: 3.0326x; 3.0326x over previous
import functools

import jax
import jax.numpy as jnp
from jax import lax
from jax.experimental import pallas as pl
from jax.experimental.pallas import tpu as pltpu
from jax.experimental.pallas import tpu_sc as plsc

B, L = 1024, 200
S = 8
C = 16
TOK = B * L
ROWS = TOK // S
D_OUT = 128


_NW = 32
_CHUNK = 128
_TPW = TOK // _NW
_CPW = _TPW // _CHUNK


_WPW = _TPW * C
_PASS = 2
_CPP = _CPW // _PASS
_WPP = _WPW // _PASS


def _sc_gather_body(table_hbm, idx_hbm, out_hbm, idx_v, g, packed, sem):
    wid = lax.axis_index("s") * 2 + lax.axis_index("c")
    pltpu.sync_copy(idx_hbm.at[pl.ds(wid * _TPW, _TPW)], idx_v)
    for p in range(_PASS):
        descs = []
        for i in range(_CPP):
            descs.append(pltpu.async_copy(
                table_hbm.at[idx_v.at[pl.ds((p * _CPP + i) * _CHUNK, _CHUNK)]],
                g.at[i], sem))
        for d in descs:
            d.wait()

        def repack(c, _):
            for r in range(_CHUNK):
                packed[pl.ds(c * (_CHUNK * C) + r * C, C)] = g[c, r]
            return 0

        lax.fori_loop(0, _CPP, repack, 0, unroll=False)
        pltpu.sync_copy(packed, out_hbm.at[pl.ds(wid * _WPW + p * _WPP, _WPP)])


def _sc_gather(table, idx_flat):
    mesh = plsc.VectorSubcoreMesh(core_axis_name="c", subcore_axis_name="s")
    kern = functools.partial(
        pl.kernel,
        mesh=mesh,
        compiler_params=pltpu.CompilerParams(use_tc_tiling_on_sc=False),
        out_type=jax.ShapeDtypeStruct((TOK * C,), jnp.float32),
        scratch_types=[
            pltpu.VMEM((_TPW,), jnp.int32),
            pltpu.VMEM((_CPP, _CHUNK, C), jnp.float32),
            pltpu.VMEM((_WPP,), jnp.float32),
            pltpu.SemaphoreType.DMA,
        ],
    )(_sc_gather_body)
    return kern(table, idx_flat)


_GRID = 32
_RB = ROWS // _GRID


def _tc_body(xv_ref, vid_ref, did_ref, p0_ref, p1_ref, p2_ref,
             de_ref, s0_ref, s1_ref, s2_ref, e8_ref, wt_ref, b_ref, out_ref):
    f32 = jnp.float32
    hi = jax.lax.Precision.HIGHEST
    mvec = (vid_ref[...] != 0).astype(f32)
    mask = jax.lax.dot(mvec, e8_ref[...], precision=hi)
    x = xv_ref[...] * mask
    i8 = lax.broadcasted_iota(jnp.int32, (_RB, 8), 1)
    i128 = lax.broadcasted_iota(jnp.int32, (_RB, 128), 1)
    pieces = []
    for k in range(S):
        ohd = (did_ref[:, k:k + 1] == i8).astype(f32)
        acc = jax.lax.dot(ohd, de_ref[...], precision=hi)
        for p_ref, s_ref in ((p0_ref, s0_ref), (p1_ref, s1_ref), (p2_ref, s2_ref)):
            ohp = (p_ref[:, k:k + 1] == i128).astype(f32)
            acc = acc + jax.lax.dot(ohp, s_ref[...], precision=hi)
        pieces.append(acc)
    x = x + jnp.concatenate(pieces, axis=1)
    y = jax.lax.dot(x, wt_ref[...], precision=hi) + b_ref[...]
    out_ref[...] = y


def _tc_embed_conv(xv, vid, did, p0, p1, p2, de8, se_z, e8, wt, bias):
    def full(shape):
        return pl.BlockSpec(shape, lambda *_: tuple(0 for _ in shape))

    return pl.pallas_call(
        _tc_body,
        grid=(_GRID,),
        in_specs=[
            pl.BlockSpec((_RB, D_OUT), lambda i: (i, 0)),
            pl.BlockSpec((_RB, S), lambda i: (i, 0)),
            pl.BlockSpec((_RB, S), lambda i: (i, 0)),
            pl.BlockSpec((_RB, S), lambda i: (i, 0)),
            pl.BlockSpec((_RB, S), lambda i: (i, 0)),
            pl.BlockSpec((_RB, S), lambda i: (i, 0)),
            full((S, C)),
            full((2 * 64, C)),
            full((2 * 64, C)),
            full((2 * 64, C)),
            full((S, D_OUT)),
            full((S * C, D_OUT)),
            full((1, D_OUT)),
        ],
        out_specs=pl.BlockSpec((_RB, D_OUT), lambda i: (i, 0)),
        out_shape=jax.ShapeDtypeStruct((ROWS, D_OUT), jnp.float32),
    )(xv, vid, did, p0, p1, p2, de8, se_z[0], se_z[1], se_z[2], e8, wt, bias)


def kernel(value, depth, position, tgt_value_emb, tgt_depth_emb,
           tgt_spatial_emb, conv_w, conv_b):
    value = value.astype(jnp.int32)
    depth = depth.astype(jnp.int32)
    position = position.astype(jnp.int32)

    xv = _sc_gather(tgt_value_emb, value.reshape(TOK)).reshape(ROWS, D_OUT)

    de8 = jnp.zeros((S, C), jnp.float32).at[1:7].set(tgt_depth_emb[1:])
    se_z = tgt_spatial_emb.at[:, 0, :].set(0.0)

    vid = value.reshape(ROWS, S)
    did = depth.reshape(ROWS, S)
    pos = position.reshape(ROWS, S, 3)
    p0, p1, p2 = pos[:, :, 0], pos[:, :, 1], pos[:, :, 2]

    e8 = jnp.repeat(jnp.eye(S, dtype=jnp.float32), C, axis=1)
    wt = conv_w.transpose(2, 1, 0).reshape(S * C, D_OUT)
    bias = conv_b.reshape(1, D_OUT)

    out = _tc_embed_conv(xv, vid, did, p0, p1, p2, de8, se_z, e8, wt, bias)
    return out.reshape(B, L // S, D_OUT)
